# baseline (speedup 1.0000x reference)
import functools

import jax
import jax.numpy as jnp
from jax import lax
from jax.experimental import pallas as pl
from jax.experimental.pallas import tpu as pltpu
from jax.experimental.pallas import tpu_sc as plsc

N = 100000
D = 128
G = 1024

NC = 2
NS = 16
NW = NC * NS

CR = 160
NG = CR // 16
CB = 32
NCHUNK = N // CR
NBASE = NCHUNK // NW
NREM = NCHUNK % NW
KMAX = NBASE + 1
GPAD = 1040
STRIPE = G // NS
HSTR = STRIPE // 4

_mesh = plsc.VectorSubcoreMesh(core_axis_name="c", subcore_axis_name="s")

_SC_OUT_TYPE = (
    jax.ShapeDtypeStruct((NC, G, D), jnp.float32),
    jax.ShapeDtypeStruct((NC, G), jnp.float32),
)
_SC_SCRATCH = (
    [pltpu.VMEM((CR, D), jnp.float32)] * 2 +
    [pltpu.VMEM((CR,), jnp.int32)] * 4 +
    [pltpu.VMEM((CB, D), jnp.float32)] * 2 +
    [pltpu.VMEM((CB,), jnp.int32)] * 2 +
    [pltpu.VMEM((NG, 16), jnp.int32)] * 2 +
    [
        pltpu.VMEM((GPAD,), jnp.float32),
        pltpu.VMEM((G // 128, 128), jnp.int32),
        pltpu.VMEM((HSTR, D), jnp.float32),
        pltpu.VMEM((STRIPE,), jnp.float32),
        pltpu.VMEM_SHARED((GPAD, D), jnp.float32),
        pltpu.VMEM_SHARED((GPAD,), jnp.float32),
        pltpu.SemaphoreType.DMA,
        pltpu.SemaphoreType.DMA,
        pltpu.SemaphoreType.DMA,
        pltpu.SemaphoreType.DMA,
    ]
)


def _sc_body(x_hbm, seg_hbm, mask_hbm,
             sums_out, cnts_out,
             xb0, xb1, sb0, sb1, mb0, mb1, cb0, cb1, ci0, ci1, fx0, fx1,
             lcnt, identbuf, stage, cstage, accum, cacc,
             gsem0, gsem1, ssem0, ssem1):
    cid = lax.axis_index("c")
    sid = lax.axis_index("s")
    wid = sid * NC + cid

    xb = (xb0, xb1)
    sb = (sb0, sb1)
    mb = (mb0, mb1)
    cb = (cb0, cb1)
    ci = (ci0, ci1)
    fx = (fx0, fx1)
    gsem = (gsem0, gsem1)
    ssem = (ssem0, ssem1)

    zvec = jnp.zeros((16,), jnp.float32)
    lane = lax.iota(jnp.int32, 16)
    gdump = jnp.full((16,), G, jnp.int32)

    @pl.loop(0, HSTR)
    def _zrow(r):
        for j in range(D // 16):
            stage[r, pl.ds(j * 16, 16)] = zvec

    for i in range(STRIPE // 16):
        cstage[pl.ds(i * 16, 16)] = zvec
    for h in range(4):
        pltpu.sync_copy(stage,
                        accum.at[pl.ds(sid * STRIPE + h * HSTR, HSTR)])
    pltpu.sync_copy(cstage, cacc.at[pl.ds(sid * STRIPE, STRIPE)])

    @pl.loop(0, GPAD // 16)
    def _zcnt(i):
        lcnt[pl.ds(i * 16, 16)] = zvec

    for bb in range(G // 128):
        for i in range(8):
            identbuf[bb, pl.ds(i * 16, 16)] = bb * 128 + i * 16 + lane

    for p in range(2):
        for i in range(CB // 16):
            ci[p][pl.ds(i * 16, 16)] = gdump

    plsc.subcore_barrier()

    start = wid * NBASE + jnp.minimum(wid, NREM)
    nch = NBASE + jnp.where(wid < NREM, 1, 0)

    def issue_gather(k, p):
        base = (start + k) * CR
        pltpu.async_copy(x_hbm.at[pl.ds(base, CR)], xb[p], gsem[p])
        pltpu.async_copy(seg_hbm.at[pl.ds(base, CR)], sb[p], gsem[p])
        pltpu.async_copy(mask_hbm.at[pl.ds(base, CR)], mb[p], gsem[p])

    def wait_gather(k, p):
        base = (start + k) * CR
        pltpu.make_async_copy(x_hbm.at[pl.ds(base, CR)], xb[p], gsem[p]).wait()
        pltpu.make_async_copy(seg_hbm.at[pl.ds(base, CR)], sb[p], gsem[p]).wait()
        pltpu.make_async_copy(mask_hbm.at[pl.ds(base, CR)], mb[p], gsem[p]).wait()

    def issue_scatter(p):
        pltpu.async_copy(cb[p], accum.at[ci[p]], ssem[p], add=True)

    def wait_scatter(p):
        pltpu.make_async_copy(cb[p], accum.at[ci[p]], ssem[p]).wait()

    issue_gather(0, 0)

    @pl.loop(0, KMAX, step=2)
    def _pipeline(ko):
        for b in range(2):
            k = ko + b
            p = b
            q = 1 - b

            @pl.when(jnp.logical_and(k >= 1, k + 1 < nch))
            def _():
                wait_scatter(q)

            @pl.when(k + 1 < nch)
            def _():
                issue_gather(k + 1, q)

            @pl.when(k < nch)
            def _():
                wait_gather(k, p)

                @pl.loop(0, NG)
                def _group(g):
                    sl = pl.ds(g * 16, 16)
                    segv = sb[p][sl]
                    maskv = mb[p][sl]
                    maskf = jnp.where(maskv == 1, 1.0, 0.0)
                    s0 = segv[0]
                    s15 = segv[15]
                    two = s15 != s0

                    acc_a = [zvec] * (D // 16)
                    acc_b = [zvec] * (D // 16)
                    wa = jnp.where(segv == s0, maskf, 0.0)
                    wb = jnp.where(segv == s0, 0.0,
                                   jnp.where(segv == s15, maskf, 0.0))
                    aw = [wa[i] for i in range(16)]
                    bw = [wb[i] for i in range(16)]
                    fast = s0 == s0
                    cnt_a = jnp.float32(0.0)
                    cnt_b = jnp.float32(0.0)
                    for i in range(16):
                        s_i = segv[i]
                        fast = jnp.logical_and(
                            fast, jnp.logical_or(s_i == s0, s_i == s15))
                        cnt_a = cnt_a + aw[i]
                        cnt_b = cnt_b + bw[i]
                    for i in range(16):
                        for j in range(D // 16):
                            v = xb[p][g * 16 + i, pl.ds(j * 16, 16)]
                            acc_a[j] = acc_a[j] + v * aw[i]
                            acc_b[j] = acc_b[j] + v * bw[i]

                    for j in range(D // 16):
                        cb[p][2 * g, pl.ds(j * 16, 16)] = acc_a[j]
                        cb[p][2 * g + 1, pl.ds(j * 16, 16)] = acc_b[j]

                    idx_a = jnp.where(fast, s0, jnp.int32(G))
                    idx_b = jnp.where(jnp.logical_and(fast, two),
                                      s15, jnp.int32(G))
                    slot = pl.ds((2 * g) // 16 * 16, 16)
                    r0 = (2 * g) % 16
                    cur = ci[p][slot]
                    cur = jnp.where(lane == r0, idx_a, cur)
                    cur = jnp.where(lane == r0 + 1, idx_b, cur)
                    ci[p][slot] = cur

                    @pl.when(fast)
                    def _fast_counts():
                        near = s15 - s0 <= 15
                        nearf = jnp.where(near, 1.0, 0.0)
                        add_a = jnp.where(lane == 0, cnt_a, 0.0)
                        add_b = jnp.where(lane == s15 - s0, cnt_b * nearf, 0.0)
                        wsl = pl.ds(s0, 16)
                        lcnt[wsl] = lcnt[wsl] + add_a + add_b

                        @pl.when(jnp.logical_and(two, jnp.logical_not(near)))
                        def _far_tail():
                            tsl = pl.ds(s15, 16)
                            lcnt[tsl] = lcnt[tsl] + jnp.where(
                                lane == 0, cnt_b, 0.0)

                    @pl.when(jnp.logical_not(fast))
                    def _slow():
                        fx[p][g, :] = jnp.where(maskv == 1, segv, gdump)
                        pltpu.sync_copy(xb[p].at[pl.ds(g * 16, 16)],
                                        accum.at[fx[p].at[g]], add=True)
                        onehot0 = jnp.where(lane == 0, 1.0, 0.0)
                        for i in range(16):
                            esl = pl.ds(segv[i], 16)
                            lcnt[esl] = lcnt[esl] + maskf[i] * onehot0

                issue_scatter(p)

    wait_scatter(0)
    wait_scatter(1)

    for b in range(G // 128):
        pltpu.sync_copy(lcnt.at[pl.ds(b * 128, 128)],
                        cacc.at[identbuf.at[b]], add=True)

    plsc.subcore_barrier()

    for h in range(4):
        s_sl = pl.ds(sid * STRIPE + h * HSTR, HSTR)
        pltpu.sync_copy(accum.at[s_sl], stage)
        pltpu.sync_copy(stage, sums_out.at[cid, s_sl])
    pltpu.sync_copy(cacc.at[pl.ds(sid * STRIPE, STRIPE)], cstage)
    pltpu.sync_copy(cstage, cnts_out.at[cid, pl.ds(sid * STRIPE, STRIPE)])


_sc_segment_sums = functools.partial(
    pl.kernel, mesh=_mesh, out_type=_SC_OUT_TYPE, scratch_types=_SC_SCRATCH,
)(_sc_body)


def _combine_body(s_ref, c_ref, o_ref):
    s = s_ref[0] + s_ref[1]
    c = c_ref[0] + c_ref[1]
    o_ref[...] = s / c


_combine = pl.pallas_call(
    _combine_body,
    out_shape=jax.ShapeDtypeStruct((G, D), jnp.float32),
)


def kernel(x, segment_ids, mask, num_segments):
    seg = segment_ids.astype(jnp.int32)
    msk = mask.astype(jnp.int32)
    sums, cnts = _sc_segment_sums(x, seg, msk)
    return _combine(sums, cnts.reshape(NC, G, 1))


# speedup vs baseline: 1.0093x; 1.0093x over previous
import functools

import jax
import jax.numpy as jnp
from jax import lax
from jax.experimental import pallas as pl
from jax.experimental.pallas import tpu as pltpu
from jax.experimental.pallas import tpu_sc as plsc

N = 100000
D = 128
G = 1024

NC = 2
NS = 16
NW = NC * NS

CR = 80
NG = CR // 16
CB = 16
NCHUNK = N // CR
NBASE = NCHUNK // NW
NREM = NCHUNK % NW
KMAX = NBASE + 1
GPAD = 1040
STRIPE = G // NS
HSTR = STRIPE // 4

_mesh = plsc.VectorSubcoreMesh(core_axis_name="c", subcore_axis_name="s")

_SC_OUT_TYPE = (
    jax.ShapeDtypeStruct((NC, G, D), jnp.float32),
    jax.ShapeDtypeStruct((NC, G), jnp.float32),
)
_SC_SCRATCH = (
    [pltpu.VMEM((CR, D), jnp.float32)] * 2 +
    [pltpu.VMEM((CR,), jnp.int32)] * 4 +
    [pltpu.VMEM((CB, D), jnp.float32)] * 2 +
    [pltpu.VMEM((CB,), jnp.int32)] * 2 +
    [pltpu.VMEM((NG, 16), jnp.int32)] * 2 +
    [
        pltpu.VMEM((GPAD,), jnp.float32),
        pltpu.VMEM((G // 128, 128), jnp.int32),
        pltpu.VMEM((HSTR, D), jnp.float32),
        pltpu.VMEM((STRIPE,), jnp.float32),
        pltpu.VMEM_SHARED((GPAD, D), jnp.float32),
        pltpu.VMEM_SHARED((GPAD,), jnp.float32),
        pltpu.SemaphoreType.DMA,
        pltpu.SemaphoreType.DMA,
        pltpu.SemaphoreType.DMA,
        pltpu.SemaphoreType.DMA,
    ]
)


def _sc_body(x_hbm, seg_hbm, mask_hbm,
             sums_out, cnts_out,
             xb0, xb1, sb0, sb1, mb0, mb1, cb0, cb1, ci0, ci1, fx0, fx1,
             lcnt, identbuf, stage, cstage, accum, cacc,
             gsem0, gsem1, ssem0, ssem1):
    cid = lax.axis_index("c")
    sid = lax.axis_index("s")
    wid = sid * NC + cid

    xb = (xb0, xb1)
    sb = (sb0, sb1)
    mb = (mb0, mb1)
    cb = (cb0, cb1)
    ci = (ci0, ci1)
    fx = (fx0, fx1)
    gsem = (gsem0, gsem1)
    ssem = (ssem0, ssem1)

    zvec = jnp.zeros((16,), jnp.float32)
    lane = lax.iota(jnp.int32, 16)
    gdump = jnp.full((16,), G, jnp.int32)

    @pl.loop(0, HSTR)
    def _zrow(r):
        for j in range(D // 16):
            stage[r, pl.ds(j * 16, 16)] = zvec

    for i in range(STRIPE // 16):
        cstage[pl.ds(i * 16, 16)] = zvec
    for h in range(4):
        pltpu.sync_copy(stage,
                        accum.at[pl.ds(sid * STRIPE + h * HSTR, HSTR)])
    pltpu.sync_copy(cstage, cacc.at[pl.ds(sid * STRIPE, STRIPE)])

    @pl.loop(0, GPAD // 16)
    def _zcnt(i):
        lcnt[pl.ds(i * 16, 16)] = zvec

    for bb in range(G // 128):
        for i in range(8):
            identbuf[bb, pl.ds(i * 16, 16)] = bb * 128 + i * 16 + lane

    for p in range(2):
        for i in range(CB // 16):
            ci[p][pl.ds(i * 16, 16)] = gdump

    plsc.subcore_barrier()

    start = wid * NBASE + jnp.minimum(wid, NREM)
    nch = NBASE + jnp.where(wid < NREM, 1, 0)

    def issue_gather(k, p):
        base = (start + k) * CR
        pltpu.async_copy(x_hbm.at[pl.ds(base, CR)], xb[p], gsem[p])
        pltpu.async_copy(seg_hbm.at[pl.ds(base, CR)], sb[p], gsem[p])
        pltpu.async_copy(mask_hbm.at[pl.ds(base, CR)], mb[p], gsem[p])

    def wait_gather(k, p):
        base = (start + k) * CR
        pltpu.make_async_copy(x_hbm.at[pl.ds(base, CR)], xb[p], gsem[p]).wait()
        pltpu.make_async_copy(seg_hbm.at[pl.ds(base, CR)], sb[p], gsem[p]).wait()
        pltpu.make_async_copy(mask_hbm.at[pl.ds(base, CR)], mb[p], gsem[p]).wait()

    def issue_scatter(p):
        pltpu.async_copy(cb[p], accum.at[ci[p]], ssem[p], add=True)

    def wait_scatter(p):
        pltpu.make_async_copy(cb[p], accum.at[ci[p]], ssem[p]).wait()

    issue_gather(0, 0)

    @pl.loop(0, KMAX, step=2)
    def _pipeline(ko):
        for b in range(2):
            k = ko + b
            p = b
            q = 1 - b

            @pl.when(jnp.logical_and(k >= 1, k + 1 < nch))
            def _():
                wait_scatter(q)

            @pl.when(k + 1 < nch)
            def _():
                issue_gather(k + 1, q)

            @pl.when(k < nch)
            def _():
                wait_gather(k, p)

                @pl.loop(0, NG)
                def _group(g):
                    sl = pl.ds(g * 16, 16)
                    segv = sb[p][sl]
                    maskv = mb[p][sl]
                    maskf = jnp.where(maskv == 1, 1.0, 0.0)
                    s0 = segv[0]
                    s15 = segv[15]
                    two = s15 != s0

                    acc_a = [zvec] * (D // 16)
                    acc_b = [zvec] * (D // 16)
                    fast = s0 == s0
                    cnt_a = jnp.float32(0.0)
                    cnt_b = jnp.float32(0.0)
                    for i in range(16):
                        s_i = segv[i]
                        m_i = maskf[i]
                        in_a = s_i == s0
                        in_b = jnp.logical_and(s_i == s15, two)
                        fast = jnp.logical_and(
                            fast, jnp.logical_or(in_a, s_i == s15))
                        a_i = jnp.where(in_a, m_i, 0.0)
                        b_i = jnp.where(in_b, m_i, 0.0)
                        cnt_a = cnt_a + a_i
                        cnt_b = cnt_b + b_i
                        for j in range(D // 16):
                            v = xb[p][g * 16 + i, pl.ds(j * 16, 16)]
                            acc_a[j] = acc_a[j] + v * a_i
                            acc_b[j] = acc_b[j] + v * b_i

                    for j in range(D // 16):
                        cb[p][2 * g, pl.ds(j * 16, 16)] = acc_a[j]
                        cb[p][2 * g + 1, pl.ds(j * 16, 16)] = acc_b[j]

                    idx_a = jnp.where(fast, s0, jnp.int32(G))
                    idx_b = jnp.where(jnp.logical_and(fast, two),
                                      s15, jnp.int32(G))
                    slot = pl.ds((2 * g) // 16 * 16, 16)
                    r0 = (2 * g) % 16
                    cur = ci[p][slot]
                    cur = jnp.where(lane == r0, idx_a, cur)
                    cur = jnp.where(lane == r0 + 1, idx_b, cur)
                    ci[p][slot] = cur

                    @pl.when(fast)
                    def _fast_counts():
                        near = s15 - s0 <= 15
                        nearf = jnp.where(near, 1.0, 0.0)
                        add_a = jnp.where(lane == 0, cnt_a, 0.0)
                        add_b = jnp.where(lane == s15 - s0, cnt_b * nearf, 0.0)
                        wsl = pl.ds(s0, 16)
                        lcnt[wsl] = lcnt[wsl] + add_a + add_b

                        @pl.when(jnp.logical_and(two, jnp.logical_not(near)))
                        def _far_tail():
                            tsl = pl.ds(s15, 16)
                            lcnt[tsl] = lcnt[tsl] + jnp.where(
                                lane == 0, cnt_b, 0.0)

                    @pl.when(jnp.logical_not(fast))
                    def _slow():
                        fx[p][g, :] = jnp.where(maskv == 1, segv, gdump)
                        pltpu.sync_copy(xb[p].at[pl.ds(g * 16, 16)],
                                        accum.at[fx[p].at[g]], add=True)
                        onehot0 = jnp.where(lane == 0, 1.0, 0.0)
                        for i in range(16):
                            esl = pl.ds(segv[i], 16)
                            lcnt[esl] = lcnt[esl] + maskf[i] * onehot0

                issue_scatter(p)

    wait_scatter(0)
    wait_scatter(1)

    for b in range(G // 128):
        pltpu.sync_copy(lcnt.at[pl.ds(b * 128, 128)],
                        cacc.at[identbuf.at[b]], add=True)

    plsc.subcore_barrier()

    for h in range(4):
        s_sl = pl.ds(sid * STRIPE + h * HSTR, HSTR)
        pltpu.sync_copy(accum.at[s_sl], stage)
        pltpu.sync_copy(stage, sums_out.at[cid, s_sl])
    pltpu.sync_copy(cacc.at[pl.ds(sid * STRIPE, STRIPE)], cstage)
    pltpu.sync_copy(cstage, cnts_out.at[cid, pl.ds(sid * STRIPE, STRIPE)])


_sc_segment_sums = functools.partial(
    pl.kernel, mesh=_mesh, out_type=_SC_OUT_TYPE, scratch_types=_SC_SCRATCH,
)(_sc_body)


def _combine_body(s_ref, c_ref, o_ref):
    s = s_ref[0] + s_ref[1]
    c = c_ref[0] + c_ref[1]
    o_ref[...] = s / c


_combine = pl.pallas_call(
    _combine_body,
    out_shape=jax.ShapeDtypeStruct((G, D), jnp.float32),
)


def kernel(x, segment_ids, mask, num_segments):
    seg = segment_ids.astype(jnp.int32)
    msk = mask.astype(jnp.int32)
    sums, cnts = _sc_segment_sums(x, seg, msk)
    return _combine(sums, cnts.reshape(NC, G, 1))


# speedup vs baseline: 1.1098x; 1.0996x over previous
import functools

import jax
import jax.numpy as jnp
from jax import lax
from jax.experimental import pallas as pl
from jax.experimental.pallas import tpu as pltpu
from jax.experimental.pallas import tpu_sc as plsc

N = 100000
D = 128
G = 1024

NC = 2
NS = 16
NW = NC * NS

CR = 160
NG = CR // 16
CB = 32
NCHUNK = N // CR
NBASE = NCHUNK // NW
NREM = NCHUNK % NW
KMAX = NBASE + 1
GPAD = 1040
STRIPE = G // NS
HSTR = STRIPE // 4

_mesh = plsc.VectorSubcoreMesh(core_axis_name="c", subcore_axis_name="s")

_SC_OUT_TYPE = (
    jax.ShapeDtypeStruct((NC, G, D), jnp.float32),
    jax.ShapeDtypeStruct((NC, G, D), jnp.float32),
)
_SC_SCRATCH = (
    [pltpu.VMEM((CR, D), jnp.float32)] * 2 +
    [pltpu.VMEM((CR,), jnp.int32)] * 4 +
    [pltpu.VMEM((CB, D), jnp.float32)] * 2 +
    [pltpu.VMEM((CB, D), jnp.float32)] * 2 +
    [pltpu.VMEM((CB,), jnp.int32)] * 2 +
    [pltpu.VMEM((NG, 16), jnp.int32)] * 2 +
    [
        pltpu.VMEM((16, D), jnp.float32),
        pltpu.VMEM((HSTR, D), jnp.float32),
        pltpu.VMEM_SHARED((GPAD, D), jnp.float32),
        pltpu.VMEM_SHARED((GPAD, D), jnp.float32),
        pltpu.SemaphoreType.DMA,
        pltpu.SemaphoreType.DMA,
        pltpu.SemaphoreType.DMA,
        pltpu.SemaphoreType.DMA,
    ]
)


def _sc_body(x_hbm, seg_hbm, mask_hbm,
             sums_out, cnts_out,
             xb0, xb1, sb0, sb1, mb0, mb1, cb0, cb1, cc0, cc1, ci0, ci1,
             fx0, fx1, dbuf, stage, accum, cacc,
             gsem0, gsem1, ssem0, ssem1):
    cid = lax.axis_index("c")
    sid = lax.axis_index("s")
    wid = sid * NC + cid

    xb = (xb0, xb1)
    sb = (sb0, sb1)
    mb = (mb0, mb1)
    cb = (cb0, cb1)
    cc = (cc0, cc1)
    ci = (ci0, ci1)
    fx = (fx0, fx1)
    gsem = (gsem0, gsem1)
    ssem = (ssem0, ssem1)

    zvec = jnp.zeros((16,), jnp.float32)
    lane = lax.iota(jnp.int32, 16)
    gdump = jnp.full((16,), G, jnp.int32)

    @pl.loop(0, HSTR)
    def _zrow(r):
        for j in range(D // 16):
            stage[r, pl.ds(j * 16, 16)] = zvec

    for h in range(4):
        hs = pl.ds(sid * STRIPE + h * HSTR, HSTR)
        pltpu.sync_copy(stage, accum.at[hs])
        pltpu.sync_copy(stage, cacc.at[hs])

    for p in range(2):
        for i in range(CB // 16):
            ci[p][pl.ds(i * 16, 16)] = gdump

    plsc.subcore_barrier()

    start = wid * NBASE + jnp.minimum(wid, NREM)
    nch = NBASE + jnp.where(wid < NREM, 1, 0)

    def issue_gather(k, p):
        base = (start + k) * CR
        pltpu.async_copy(x_hbm.at[pl.ds(base, CR)], xb[p], gsem[p])
        pltpu.async_copy(seg_hbm.at[pl.ds(base, CR)], sb[p], gsem[p])
        pltpu.async_copy(mask_hbm.at[pl.ds(base, CR)], mb[p], gsem[p])

    def wait_gather(k, p):
        base = (start + k) * CR
        pltpu.make_async_copy(x_hbm.at[pl.ds(base, CR)], xb[p], gsem[p]).wait()
        pltpu.make_async_copy(seg_hbm.at[pl.ds(base, CR)], sb[p], gsem[p]).wait()
        pltpu.make_async_copy(mask_hbm.at[pl.ds(base, CR)], mb[p], gsem[p]).wait()

    def issue_scatter(p):
        pltpu.async_copy(cb[p], accum.at[ci[p]], ssem[p], add=True)
        pltpu.async_copy(cc[p], cacc.at[ci[p]], ssem[p], add=True)

    def wait_scatter(p):
        pltpu.make_async_copy(cb[p], accum.at[ci[p]], ssem[p]).wait()
        pltpu.make_async_copy(cc[p], cacc.at[ci[p]], ssem[p]).wait()

    issue_gather(0, 0)

    @pl.loop(0, KMAX, step=2)
    def _pipeline(ko):
        for b in range(2):
            k = ko + b
            p = b
            q = 1 - b

            @pl.when(jnp.logical_and(k >= 1, k + 1 < nch))
            def _():
                wait_scatter(q)

            @pl.when(k + 1 < nch)
            def _():
                issue_gather(k + 1, q)

            @pl.when(k < nch)
            def _():
                wait_gather(k, p)

                @pl.loop(0, NG)
                def _group(g):
                    sl = pl.ds(g * 16, 16)
                    segv = sb[p][sl]
                    maskv = mb[p][sl]
                    maskf = jnp.where(maskv == 1, 1.0, 0.0)
                    s0 = segv[0]
                    s15 = segv[15]
                    two = s15 != s0

                    wa = jnp.where(segv == s0, maskf, 0.0)
                    wb = jnp.where(segv == s0, 0.0,
                                   jnp.where(segv == s15, maskf, 0.0))
                    cc[p][2 * g, pl.ds(0, 16)] = wa
                    cc[p][2 * g + 1, pl.ds(0, 16)] = wb

                    acc_a = [zvec] * (D // 16)
                    acc_b = [zvec] * (D // 16)
                    fast = s0 == s0
                    for i in range(16):
                        s_i = segv[i]
                        m_i = maskf[i]
                        in_a = s_i == s0
                        in_b = jnp.logical_and(s_i == s15, two)
                        fast = jnp.logical_and(
                            fast, jnp.logical_or(in_a, s_i == s15))
                        a_i = jnp.where(in_a, m_i, 0.0)
                        b_i = jnp.where(in_b, m_i, 0.0)
                        for j in range(D // 16):
                            v = xb[p][g * 16 + i, pl.ds(j * 16, 16)]
                            acc_a[j] = acc_a[j] + v * a_i
                            acc_b[j] = acc_b[j] + v * b_i

                    for j in range(D // 16):
                        cb[p][2 * g, pl.ds(j * 16, 16)] = acc_a[j]
                        cb[p][2 * g + 1, pl.ds(j * 16, 16)] = acc_b[j]

                    idx_a = jnp.where(fast, s0, jnp.int32(G))
                    idx_b = jnp.where(jnp.logical_and(fast, two),
                                      s15, jnp.int32(G))
                    slot = pl.ds((2 * g) // 16 * 16, 16)
                    r0 = (2 * g) % 16
                    cur = ci[p][slot]
                    cur = jnp.where(lane == r0, idx_a, cur)
                    cur = jnp.where(lane == r0 + 1, idx_b, cur)
                    ci[p][slot] = cur

                    @pl.when(jnp.logical_not(fast))
                    def _slow():
                        fx[p][g, pl.ds(0, 16)] = jnp.where(
                            maskv == 1, segv, gdump)
                        for i in range(16):
                            dbuf[i, pl.ds(0, 16)] = jnp.where(
                                lane == i, maskf, 0.0)
                        pltpu.sync_copy(xb[p].at[pl.ds(g * 16, 16)],
                                        accum.at[fx[p].at[g]], add=True)
                        pltpu.sync_copy(dbuf, cacc.at[fx[p].at[g]], add=True)

                issue_scatter(p)

    wait_scatter(0)
    wait_scatter(1)

    plsc.subcore_barrier()

    for h in range(4):
        s_sl = pl.ds(sid * STRIPE + h * HSTR, HSTR)
        pltpu.sync_copy(accum.at[s_sl], stage)
        pltpu.sync_copy(stage, sums_out.at[cid, s_sl])
    for h in range(4):
        s_sl = pl.ds(sid * STRIPE + h * HSTR, HSTR)
        pltpu.sync_copy(cacc.at[s_sl], stage)
        pltpu.sync_copy(stage, cnts_out.at[cid, s_sl])


_sc_segment_sums = functools.partial(
    pl.kernel, mesh=_mesh, out_type=_SC_OUT_TYPE, scratch_types=_SC_SCRATCH,
)(_sc_body)


def _combine_body(s_ref, c_ref, o_ref):
    s = s_ref[0] + s_ref[1]
    col = lax.broadcasted_iota(jnp.int32, (G, D), 1)
    cm = jnp.where(col < 16, 1.0, 0.0)
    c = jnp.sum((c_ref[0] + c_ref[1]) * cm, axis=-1, keepdims=True)
    o_ref[...] = s / c


_combine = pl.pallas_call(
    _combine_body,
    out_shape=jax.ShapeDtypeStruct((G, D), jnp.float32),
)


def kernel(x, segment_ids, mask, num_segments):
    seg = segment_ids.astype(jnp.int32)
    msk = mask.astype(jnp.int32)
    sums, cnts = _sc_segment_sums(x, seg, msk)
    return _combine(sums, cnts)
